# Initial kernel scaffold; baseline (speedup 1.0000x reference)
import jax
import jax.numpy as jnp
from jax.experimental import pallas as pl


def kernel(out_ids, tree_ids, ctx_ids, out_table, tree_table, ctx_table):
    raise NotImplementedError("write your pallas kernel here")


# speedup vs baseline: 5.9842x; 5.9842x over previous
import functools

import jax
import jax.numpy as jnp
from jax import lax
from jax.experimental import pallas as pl
from jax.experimental.pallas import tpu as pltpu
from jax.experimental.pallas import tpu_sc as plsc

B, L = 4096, 50
N = B * L
D = 64
NC, NS = 2, 16
NW = NC * NS
ROWS_PER_W = N // NW
C = 128
NCHUNK = ROWS_PER_W // C

_mesh = plsc.VectorSubcoreMesh(core_axis_name="c", subcore_axis_name="s")


@functools.partial(
    pl.kernel,
    mesh=_mesh,
    out_type=jax.ShapeDtypeStruct((N, D), jnp.float32),
    compiler_params=pltpu.CompilerParams(use_tc_tiling_on_sc=False),
    scratch_types=[
        pltpu.VMEM((C,), jnp.int32),
        pltpu.VMEM((C,), jnp.int32),
        pltpu.VMEM((C,), jnp.int32),
        pltpu.VMEM((C, D), jnp.float32),
        pltpu.VMEM((C, D), jnp.float32),
        pltpu.VMEM((C, D), jnp.float32),
        pltpu.SemaphoreType.DMA,
        pltpu.SemaphoreType.DMA,
        pltpu.SemaphoreType.DMA,
    ],
)
def _triple_embed(oid, tid, cid, t1, t2, t3, out,
                  i1, i2, i3, b1, b2, b3, s1, s2, s3):
    wid = lax.axis_index("s") * NC + lax.axis_index("c")
    wbase = wid * ROWS_PER_W

    def chunk(c, carry):
        base = wbase + c * C
        pltpu.sync_copy(oid.at[pl.ds(base, C)], i1)
        pltpu.sync_copy(tid.at[pl.ds(base, C)], i2)
        pltpu.sync_copy(cid.at[pl.ds(base, C)], i3)
        cp1 = pltpu.async_copy(t1.at[i1], b1, s1)
        cp2 = pltpu.async_copy(t2.at[i2], b2, s2)
        cp3 = pltpu.async_copy(t3.at[i3], b3, s3)
        cp1.wait()
        cp2.wait()
        cp3.wait()

        def row(r, cc):
            for j in range(D // 16):
                sl = pl.ds(j * 16, 16)
                b1[r, sl] = b1[r, sl] + b2[r, sl] + b3[r, sl]
            return cc

        lax.fori_loop(0, C, row, 0)
        pltpu.sync_copy(b1, out.at[pl.ds(base, C)])
        return carry

    lax.fori_loop(0, NCHUNK, chunk, 0)


def kernel(out_ids, tree_ids, ctx_ids, out_table, tree_table, ctx_table):
    oid = out_ids.reshape(-1).astype(jnp.int32)
    tid = tree_ids.reshape(-1).astype(jnp.int32)
    cid = ctx_ids.reshape(-1).astype(jnp.int32)
    res = _triple_embed(oid, tid, cid, out_table, tree_table, ctx_table)
    return res.reshape(B, L, D)


# speedup vs baseline: 8.0666x; 1.3480x over previous
import functools

import jax
import jax.numpy as jnp
from jax import lax
from jax.experimental import pallas as pl
from jax.experimental.pallas import tpu as pltpu
from jax.experimental.pallas import tpu_sc as plsc

B, L = 4096, 50
N = B * L
D = 64
NC, NS = 2, 16
NW = NC * NS
ROWS_PER_W = N // NW
C = 128
NCHUNK = ROWS_PER_W // C

_mesh = plsc.VectorSubcoreMesh(core_axis_name="c", subcore_axis_name="s")


@functools.partial(
    pl.kernel,
    mesh=_mesh,
    out_type=jax.ShapeDtypeStruct((N, D), jnp.float32),
    compiler_params=pltpu.CompilerParams(use_tc_tiling_on_sc=False),
    scratch_types=[
        pltpu.VMEM((NCHUNK, C), jnp.int32),
        pltpu.VMEM((NCHUNK, C), jnp.int32),
        pltpu.VMEM((NCHUNK, C), jnp.int32),
        pltpu.VMEM((2, C, D), jnp.float32),
        pltpu.VMEM((2, C, D), jnp.float32),
        pltpu.VMEM((2, C, D), jnp.float32),
        pltpu.SemaphoreType.DMA,
        pltpu.SemaphoreType.DMA,
        pltpu.SemaphoreType.DMA,
        pltpu.SemaphoreType.DMA,
        pltpu.SemaphoreType.DMA,
        pltpu.SemaphoreType.DMA,
    ],
)
def _triple_embed(oid, tid, cid, t1, t2, t3, out,
                  i1, i2, i3, b1, b2, b3, sa1, sa2, sa3, sb1, sb2, sb3):
    wid = lax.axis_index("s") * NC + lax.axis_index("c")
    wbase = wid * ROWS_PER_W

    pltpu.sync_copy(oid.at[wid], i1)
    pltpu.sync_copy(tid.at[wid], i2)
    pltpu.sync_copy(cid.at[wid], i3)

    sems = ((sa1, sa2, sa3), (sb1, sb2, sb3))

    def fire(c, k):
        s1, s2, s3 = sems[k]
        pltpu.async_copy(t1.at[i1.at[c]], b1.at[k], s1)
        pltpu.async_copy(t2.at[i2.at[c]], b2.at[k], s2)
        pltpu.async_copy(t3.at[i3.at[c]], b3.at[k], s3)

    def drain(c, k):
        s1, s2, s3 = sems[k]
        pltpu.make_async_copy(t1.at[i1.at[c]], b1.at[k], s1).wait()
        pltpu.make_async_copy(t2.at[i2.at[c]], b2.at[k], s2).wait()
        pltpu.make_async_copy(t3.at[i3.at[c]], b3.at[k], s3).wait()

        def row(r, cc):
            for j in range(D // 16):
                sl = pl.ds(j * 16, 16)
                b1[k, r, sl] = b1[k, r, sl] + b2[k, r, sl] + b3[k, r, sl]
            return cc

        lax.fori_loop(0, C, row, 0)
        pltpu.sync_copy(b1.at[k], out.at[pl.ds(wbase + c * C, C)])

    fire(0, 0)

    def body(h, carry):
        ca = 2 * h
        fire(ca + 1, 1)
        drain(ca, 0)

        @pl.when(ca + 2 < NCHUNK)
        def _():
            fire(ca + 2, 0)

        drain(ca + 1, 1)
        return carry

    lax.fori_loop(0, NCHUNK // 2, body, 0)


def kernel(out_ids, tree_ids, ctx_ids, out_table, tree_table, ctx_table):
    oid = out_ids.reshape(NW, NCHUNK, C).astype(jnp.int32)
    tid = tree_ids.reshape(NW, NCHUNK, C).astype(jnp.int32)
    cid = ctx_ids.reshape(NW, NCHUNK, C).astype(jnp.int32)
    res = _triple_embed(oid, tid, cid, out_table, tree_table, ctx_table)
    return res.reshape(B, L, D)
